# baseline (speedup 1.0000x reference)
import jax
import jax.numpy as jnp
from jax import lax
from jax.experimental import pallas as pl
from jax.experimental.pallas import tpu as pltpu
from jax.experimental.pallas import tpu_sc as plsc

N = 10000
E = 160000
H = 128
R = 64
TAB = 256
NG = 9
EP = 163840
EROWSP = EP // 128
CH_ROW0 = (0, 384, 768)
CH_ROWS = (384, 384, 512)
NBLK = 625
F32 = jnp.float32
HIGH = lax.Precision.HIGHEST


def _embed_body(offs, x_ref, as_ref, tab_ref, wl_ref, wr_ref, b_ref,
                cs_ref, cd_ref):
    cols = lax.broadcasted_iota(jnp.int32, (1, TAB), 1)
    oh = (x_ref[...] == cols).astype(F32)
    asv = as_ref[...]
    for i in range(9):
        oh = oh + ((asv[:, i:i + 1] + offs[i]) == cols).astype(F32)
    z = jnp.dot(oh, tab_ref[...], preferred_element_type=F32)
    dn = (((1,), (1,)), ((), ()))
    cs_ref[...] = lax.dot_general(z, wl_ref[...], dn,
                                  preferred_element_type=F32)
    cd_ref[...] = lax.dot_general(z, wr_ref[...], dn,
                                  preferred_element_type=F32) + b_ref[...]


def _embed(offs, x2d, as2d, tab, wl, wr, b2):
    blk = 1000
    import functools
    return pl.pallas_call(
        functools.partial(_embed_body, offs),
        grid=(N // blk,),
        in_specs=[
            pl.BlockSpec((blk, 1), lambda i: (i, 0)),
            pl.BlockSpec((blk, 9), lambda i: (i, 0)),
            pl.BlockSpec((TAB, H), lambda i: (0, 0)),
            pl.BlockSpec((H, H), lambda i: (0, 0)),
            pl.BlockSpec((H, H), lambda i: (0, 0)),
            pl.BlockSpec((1, H), lambda i: (0, 0)),
        ],
        out_specs=[
            pl.BlockSpec((blk, H), lambda i: (i, 0)),
            pl.BlockSpec((blk, H), lambda i: (i, 0)),
        ],
        out_shape=[
            jax.ShapeDtypeStruct((N, H), F32),
            jax.ShapeDtypeStruct((N, H), F32),
        ],
    )(x2d, as2d, tab, wl, wr, b2)


def _gather_body(row0, hi, lo, src_hbm, dst_hbm, cs_hbm, cd_hbm,
                 csg_hbm, cdg_hbm,
                 idxs, idxd, ba0, ba1, bb0, bb1, sa0, sa1, sb0, sb1):
    cid = lax.axis_index("c")
    sid = lax.axis_index("s")
    wid = sid * 2 + cid
    ishi = wid < 16
    out0 = jnp.where(ishi, wid * hi, 16 * hi + (wid - 16) * lo)
    r0 = row0 + out0
    nr = jnp.where(ishi, hi, lo)

    @pl.when(ishi)
    def _():
        pltpu.sync_copy(src_hbm.at[pl.ds(r0, hi), :],
                        idxs.at[pl.ds(0, hi), :])
        pltpu.sync_copy(dst_hbm.at[pl.ds(r0, hi), :],
                        idxd.at[pl.ds(0, hi), :])

    @pl.when(jnp.logical_not(ishi))
    def _():
        pltpu.sync_copy(src_hbm.at[pl.ds(r0, lo), :],
                        idxs.at[pl.ds(0, lo), :])
        pltpu.sync_copy(dst_hbm.at[pl.ds(r0, lo), :],
                        idxd.at[pl.ds(0, lo), :])

    def start(r, ba, bb, sa, sb):
        pltpu.make_async_copy(cs_hbm.at[idxs.at[r]], ba, sa).start()
        pltpu.make_async_copy(cd_hbm.at[idxd.at[r]], bb, sb).start()

    def finish(r, ba, bb, sa, sb):
        pltpu.make_async_copy(cs_hbm.at[idxs.at[r]], ba, sa).wait()
        pltpu.make_async_copy(cd_hbm.at[idxd.at[r]], bb, sb).wait()
        e0 = (out0 + r) * 128
        pltpu.sync_copy(ba, csg_hbm.at[pl.ds(e0, 128), :])
        pltpu.sync_copy(bb, cdg_hbm.at[pl.ds(e0, 128), :])

    start(0, ba0, bb0, sa0, sb0)

    def pair(k, carry):
        start(2 * k + 1, ba1, bb1, sa1, sb1)
        finish(2 * k, ba0, bb0, sa0, sb0)
        start(2 * k + 2, ba0, bb0, sa0, sb0)
        finish(2 * k + 1, ba1, bb1, sa1, sb1)
        return carry

    lax.fori_loop(0, nr // 2 - 1, pair, 0)
    start(nr - 1, ba1, bb1, sa1, sb1)
    finish(nr - 2, ba0, bb0, sa0, sb0)
    finish(nr - 1, ba1, bb1, sa1, sb1)


def _sc_gather(chunk, cs, cd, src2d, dst2d):
    import functools
    rows = CH_ROWS[chunk]
    hi = 16
    lo = (rows - 16 * hi) // 16
    f = pl.kernel(
        functools.partial(_gather_body, CH_ROW0[chunk], hi, lo),
        out_type=[
            jax.ShapeDtypeStruct((rows * 128, H), F32),
            jax.ShapeDtypeStruct((rows * 128, H), F32),
        ],
        mesh=plsc.VectorSubcoreMesh(core_axis_name="c", subcore_axis_name="s",
                                    num_cores=2, num_subcores=16),
        scratch_types=[
            pltpu.VMEM((16, 128), jnp.int32),
            pltpu.VMEM((16, 128), jnp.int32),
            pltpu.VMEM((128, H), F32),
            pltpu.VMEM((128, H), F32),
            pltpu.VMEM((128, H), F32),
            pltpu.VMEM((128, H), F32),
            pltpu.SemaphoreType.DMA,
            pltpu.SemaphoreType.DMA,
            pltpu.SemaphoreType.DMA,
            pltpu.SemaphoreType.DMA,
        ],
    )
    return f(src2d, dst2d, cs, cd)


def _edge_body(nreal, rbf_ref, csg_ref, cdg_ref, vt_ref, w1_ref, b1_ref,
               w2_ref, b2_ref, w3_ref, b3_ref, msg_ref):
    def compute():
        c = csg_ref[...] + cdg_ref[...]
        rbf = rbf_ref[...]
        dn = (((1,), (1,)), ((), ()))
        u1 = (lax.dot_general(rbf, w1_ref[...], dn,
                              preferred_element_type=F32) + b1_ref[...]) * c
        u2 = (lax.dot_general(rbf, w2_ref[...], dn,
                              preferred_element_type=F32) + b2_ref[...]) * c
        u3 = (lax.dot_general(rbf, w3_ref[...], dn,
                              preferred_element_type=F32) + b3_ref[...]) * c
        eye3 = jnp.eye(3, dtype=F32)
        vv = lax.dot_general(vt_ref[...], eye3, (((0,), (0,)), ((), ())),
                             preferred_element_type=F32, precision=HIGH)
        v0, v1, v2 = vv[:, 0:1], vv[:, 1:2], vv[:, 2:3]
        msg_ref[0] = u1
        msg_ref[1] = u2 * v0
        msg_ref[2] = u2 * v1
        msg_ref[3] = u2 * v2
        msg_ref[4] = u3 * (v0 * v0 - v2 * v2)
        msg_ref[5] = u3 * (v1 * v1 - v2 * v2)
        msg_ref[6] = u3 * (v0 * v1)
        msg_ref[7] = u3 * (v0 * v2)
        msg_ref[8] = u3 * (v1 * v2)

    if nreal is None:
        compute()
    else:
        @pl.when(pl.program_id(0) < nreal)
        def _():
            compute()

        @pl.when(pl.program_id(0) >= nreal)
        def _():
            z = jnp.zeros((msg_ref.shape[1], H), F32)
            for k in range(NG):
                msg_ref[k] = z


def _edge_msg(chunk, rbf, csg, cdg, vt, w1, b1, w2, b2, w3, b3):
    blk = 256
    ec = CH_ROWS[chunk] * 128
    off = CH_ROW0[chunk] * 128 // blk
    nblks = ec // blk
    nreal = min(nblks, NBLK - off) if off + nblks > NBLK else None
    import functools
    wspec = pl.BlockSpec((H, R), lambda i: (0, 0))
    bspec = pl.BlockSpec((1, H), lambda i: (0, 0))
    return pl.pallas_call(
        functools.partial(_edge_body, nreal),
        grid=(nblks,),
        in_specs=[
            pl.BlockSpec((blk, R),
                         lambda i: (jnp.minimum(i + off, NBLK - 1), 0)),
            pl.BlockSpec((blk, H), lambda i: (i, 0)),
            pl.BlockSpec((blk, H), lambda i: (i, 0)),
            pl.BlockSpec((3, blk), lambda i: (0, i + off)),
            wspec, bspec, wspec, bspec, wspec, bspec,
        ],
        out_specs=pl.BlockSpec((NG, blk, H), lambda i: (0, i, 0)),
        out_shape=jax.ShapeDtypeStruct((NG, ec, H), F32),
    )(rbf, csg, cdg, vt, w1, b1, w2, b2, w3, b3)


def _scatter_body(chunk, dst_hbm, msg_hbm, p_hbm,
                  acc, idx, idx4, zbuf, b0, b1, s0, s1):
    cid = lax.axis_index("c")
    sid = lax.axis_index("s")
    row0 = CH_ROW0[chunk]
    rows = CH_ROWS[chunk]
    nbt = rows // 16
    g4a = rows - (rows // 2 // 128) * 128
    g4b = rows - g4a
    nbt4a = g4a // 16
    nbt4b = g4b // 16
    r0 = row0 + sid * nbt

    z16 = jnp.zeros((16,), F32)

    def zrow(r, carry):
        for cc in range(8):
            zbuf[r, pl.ds(cc * 16, 16)] = z16
        return carry

    lax.fori_loop(0, 64, zrow, 0)

    pltpu.sync_copy(dst_hbm.at[pl.ds(r0, nbt), :],
                    idx.at[pl.ds(0, nbt), :])

    @pl.when(cid == 0)
    def _():
        pltpu.sync_copy(dst_hbm.at[pl.ds(row0 + sid * nbt4a, nbt4a), :],
                        idx4.at[pl.ds(0, nbt4a), :])

    @pl.when(cid == 1)
    def _():
        pltpu.sync_copy(
            dst_hbm.at[pl.ds(row0 + g4a + sid * nbt4b, nbt4b), :],
            idx4.at[pl.ds(0, nbt4b), :])

    def zero_acc():
        for j in range(9):
            pltpu.sync_copy(zbuf, acc.at[pl.ds(sid * 624 + j * 64, 64), :])
        pltpu.sync_copy(zbuf.at[pl.ds(0, 48), :],
                        acc.at[pl.ds(sid * 624 + 576, 48), :])

        @pl.when(sid == 15)
        def _():
            pltpu.sync_copy(zbuf.at[pl.ds(0, 16), :],
                            acc.at[pl.ds(9984, 16), :])

    def dump(gout):
        pltpu.sync_copy(acc.at[pl.ds(sid * 624, 624), :],
                        p_hbm.at[gout, pl.ds(sid * 624, 624), :])

        @pl.when(sid == 15)
        def _():
            pltpu.sync_copy(acc.at[pl.ds(9984, 16), :],
                            p_hbm.at[gout, pl.ds(9984, 16), :])

    def proc(g, gout, base, nb, idxbuf):
        zero_acc()
        plsc.subcore_barrier()

        def start(r, buf, sem):
            e0 = (base + r) * 128
            pltpu.make_async_copy(msg_hbm.at[g, pl.ds(e0, 128), :],
                                  buf, sem).start()

        def finish(r, buf, sem):
            e0 = (base + r) * 128
            pltpu.make_async_copy(msg_hbm.at[g, pl.ds(e0, 128), :],
                                  buf, sem).wait()
            pltpu.sync_copy(buf, acc.at[idxbuf.at[r]], add=True)

        start(0, b0, s0)

        def pair(k, carry):
            start(2 * k + 1, b1, s1)
            finish(2 * k, b0, s0)
            start(2 * k + 2, b0, s0)
            finish(2 * k + 1, b1, s1)
            return carry

        lax.fori_loop(0, nb // 2 - 1, pair, 0)
        start(nb - 1, b1, s1)
        finish(nb - 2, b0, s0)
        finish(nb - 1, b1, s1)

        plsc.subcore_barrier()
        dump(gout)
        plsc.subcore_barrier()

    @pl.when(cid == 0)
    def _():
        for g in (0, 1, 2, 3):
            proc(g, g, sid * nbt, nbt, idx)
        proc(4, 4, sid * nbt4a, nbt4a, idx4)

    @pl.when(cid == 1)
    def _():
        for g in (5, 6, 7, 8):
            proc(g, g, sid * nbt, nbt, idx)
        proc(4, 9, g4a + sid * nbt4b, nbt4b, idx4)


def _sc_scatter(chunk, msg, dst2d):
    import functools
    f = pl.kernel(
        functools.partial(_scatter_body, chunk),
        out_type=jax.ShapeDtypeStruct((NG + 1, N, H), F32),
        mesh=plsc.VectorSubcoreMesh(core_axis_name="c", subcore_axis_name="s",
                                    num_cores=2, num_subcores=16),
        scratch_types=[
            pltpu.VMEM_SHARED((N, H), F32),
            pltpu.VMEM((32, 128), jnp.int32),
            pltpu.VMEM((16, 128), jnp.int32),
            pltpu.VMEM((64, H), F32),
            pltpu.VMEM((128, H), F32),
            pltpu.VMEM((128, H), F32),
            pltpu.SemaphoreType.DMA,
            pltpu.SemaphoreType.DMA,
        ],
    )
    return f(dst2d, msg)


def _post_body(p0_ref, p1_ref, p2_ref, lng_ref, lnb_ref, ws1_ref, ws1b_ref,
               ws2r_ref, b20_ref, b21_ref, b22_ref,
               wt0_ref, wt1_ref, wt2_ref, out_ref):
    p0 = p0_ref[0] + p1_ref[0] + p2_ref[0]
    pa0 = p0_ref[1] + p1_ref[1] + p2_ref[1]
    pa1 = p0_ref[2] + p1_ref[2] + p2_ref[2]
    pa2 = p0_ref[3] + p1_ref[3] + p2_ref[3]
    pt0 = (p0_ref[4] + p1_ref[4] + p2_ref[4]
           + p0_ref[9] + p1_ref[9] + p2_ref[9])
    pt1 = p0_ref[5] + p1_ref[5] + p2_ref[5]
    p01 = p0_ref[6] + p1_ref[6] + p2_ref[6]
    p02 = p0_ref[7] + p1_ref[7] + p2_ref[7]
    p12 = p0_ref[8] + p1_ref[8] + p2_ref[8]

    s00 = (2.0 * pt0 - pt1) / 3.0
    s11 = (2.0 * pt1 - pt0) / 3.0
    s22 = -(pt0 + pt1) / 3.0
    fro = (3.0 * p0 * p0
           + 2.0 * (pa0 * pa0 + pa1 * pa1 + pa2 * pa2)
           + s00 * s00 + s11 * s11 + s22 * s22
           + 2.0 * (p01 * p01 + p02 * p02 + p12 * p12))

    mu = jnp.mean(fro, axis=-1, keepdims=True)
    var = jnp.mean((fro - mu) ** 2, axis=-1, keepdims=True)
    y = lng_ref[...] * (fro - mu) * lax.rsqrt(var + 1e-5) + lnb_ref[...]

    dn = (((1,), (1,)), ((), ()))

    def silu(t):
        return t * lax.logistic(t)

    h1 = silu(lax.dot_general(y, ws1_ref[...], dn, preferred_element_type=F32)
              + ws1b_ref[...])
    w2all = ws2r_ref[...]
    w20 = w2all[:, 0, :]
    w21 = w2all[:, 1, :]
    w22 = w2all[:, 2, :]
    n0 = silu(lax.dot_general(h1, w20, dn, preferred_element_type=F32)
              + b20_ref[...])
    n1 = silu(lax.dot_general(h1, w21, dn, preferred_element_type=F32)
              + b21_ref[...])
    n2 = silu(lax.dot_general(h1, w22, dn, preferred_element_type=F32)
              + b22_ref[...])

    a = lax.dot_general(p0, wt0_ref[...], dn, preferred_element_type=F32) * n0
    wt1 = wt1_ref[...]
    b0 = lax.dot_general(pa0, wt1, dn, preferred_element_type=F32) * n1
    b1 = lax.dot_general(pa1, wt1, dn, preferred_element_type=F32) * n1
    b2 = lax.dot_general(pa2, wt1, dn, preferred_element_type=F32) * n1
    wt2 = wt2_ref[...]
    t0 = lax.dot_general(pt0, wt2, dn, preferred_element_type=F32) * n2
    t1 = lax.dot_general(pt1, wt2, dn, preferred_element_type=F32) * n2
    q01 = lax.dot_general(p01, wt2, dn, preferred_element_type=F32) * n2
    q02 = lax.dot_general(p02, wt2, dn, preferred_element_type=F32) * n2
    q12 = lax.dot_general(p12, wt2, dn, preferred_element_type=F32) * n2

    o00 = (2.0 * t0 - t1) / 3.0
    o11 = (2.0 * t1 - t0) / 3.0
    o22 = -(t0 + t1) / 3.0
    out_ref[0] = a + o00
    out_ref[1] = -b2 + q01
    out_ref[2] = b1 + q02
    out_ref[3] = b2 + q01
    out_ref[4] = a + o11
    out_ref[5] = -b0 + q12
    out_ref[6] = -b1 + q02
    out_ref[7] = b0 + q12
    out_ref[8] = a + o22


def _node_post(pp0, pp1, pp2, ln_g, ln_b, ws1, ws1b, ws2r, b20, b21, b22,
               wt0, wt1, wt2):
    blk = 1000
    hh = pl.BlockSpec((H, H), lambda i: (0, 0))
    bias = pl.BlockSpec((1, H), lambda i: (0, 0))
    pspec = pl.BlockSpec((NG + 1, blk, H), lambda i: (0, i, 0))

    return pl.pallas_call(
        _post_body,
        grid=(N // blk,),
        in_specs=[
            pspec, pspec, pspec,
            bias, bias,
            pl.BlockSpec((2 * H, H), lambda i: (0, 0)),
            pl.BlockSpec((1, 2 * H), lambda i: (0, 0)),
            pl.BlockSpec((H, 3, 2 * H), lambda i: (0, 0, 0)),
            bias, bias, bias,
            hh, hh, hh,
        ],
        out_specs=pl.BlockSpec((NG, blk, H), lambda i: (0, i, 0)),
        out_shape=jax.ShapeDtypeStruct((NG, N, H), F32),
    )(pp0, pp1, pp2, ln_g, ln_b, ws1, ws1b,
      ws2r, b20, b21, b22, wt0, wt1, wt2)


def kernel(x, atom_scalar, edge_index, dist, vec_norm, rbf, emb,
           ae0, ae1, ae2, ae3, ae4, ae5, ae6, ae7, ae8,
           W1, b1, W2, b2, W3, b3, emb2_W, emb2_b, ln_g, ln_b,
           Wt0, Wt1, Wt2, Ws1_W, Ws1_b, Ws2_W, Ws2_b):
    del dist

    aes = [ae0, ae1, ae2, ae3, ae4, ae5, ae6, ae7, ae8]
    dims = [a.shape[0] for a in aes]
    tab = jnp.concatenate(
        [emb] + aes + [jnp.zeros((TAB - 128 - sum(dims), H), F32)], axis=0)
    offs, o = [], 128
    for d in dims:
        offs.append(o)
        o += d

    wl = emb2_W[:, :H]
    wr = emb2_W[:, H:]
    fill = jnp.broadcast_to(
        jnp.arange(EP - E, dtype=jnp.int32)[None, :] % N, (2, EP - E))
    ei = jnp.concatenate([edge_index.astype(jnp.int32), fill], axis=1)
    src2d = ei[0].reshape(EROWSP, 128)
    dst2d = ei[1].reshape(EROWSP, 128)
    vt = jnp.pad(vec_norm.T, ((0, 0), (0, EP - E)))

    cs, cd = _embed(offs, x.astype(jnp.int32).reshape(N, 1),
                    atom_scalar.astype(jnp.int32), tab, wl, wr,
                    emb2_b.reshape(1, H))
    bias1 = (b1.reshape(1, H), b2.reshape(1, H), b3.reshape(1, H))
    pps = []
    for ch in range(3):
        csg, cdg = _sc_gather(ch, cs, cd, src2d, dst2d)
        msg = _edge_msg(ch, rbf, csg, cdg, vt, W1, bias1[0], W2, bias1[1],
                        W3, bias1[2])
        pps.append(_sc_scatter(ch, msg, dst2d))
    out9 = _node_post(pps[0], pps[1], pps[2],
                      ln_g.reshape(1, H), ln_b.reshape(1, H),
                      Ws1_W, Ws1_b.reshape(1, 2 * H),
                      Ws2_W.reshape(H, 3, 2 * H),
                      Ws2_b[0::3].reshape(1, H),
                      Ws2_b[1::3].reshape(1, H),
                      Ws2_b[2::3].reshape(1, H),
                      Wt0, Wt1, Wt2)
    return jnp.transpose(out9, (1, 2, 0)).reshape(N, H, 3, 3)


# speedup vs baseline: 1.1090x; 1.1090x over previous
import jax
import jax.numpy as jnp
from jax import lax
from jax.experimental import pallas as pl
from jax.experimental.pallas import tpu as pltpu
from jax.experimental.pallas import tpu_sc as plsc

N = 10000
E = 160000
H = 128
R = 64
TAB = 256
NG = 9
EP = 163840
EROWSP = EP // 128
EC = EP // 2
NBLK = 250
F32 = jnp.float32
HIGH = lax.Precision.HIGHEST


def _embed_body(offs, x_ref, as_ref, tab_ref, wl_ref, wr_ref, b_ref,
                cs_ref, cd_ref):
    cols = lax.broadcasted_iota(jnp.int32, (1, TAB), 1)
    oh = (x_ref[...] == cols).astype(F32)
    asv = as_ref[...]
    for i in range(9):
        oh = oh + ((asv[:, i:i + 1] + offs[i]) == cols).astype(F32)
    z = jnp.dot(oh, tab_ref[...], preferred_element_type=F32)
    dn = (((1,), (1,)), ((), ()))
    cs_ref[...] = lax.dot_general(z, wl_ref[...], dn,
                                  preferred_element_type=F32)
    cd_ref[...] = lax.dot_general(z, wr_ref[...], dn,
                                  preferred_element_type=F32) + b_ref[...]


def _embed(offs, x2d, as2d, tab, wl, wr, b2):
    blk = 1000
    import functools
    return pl.pallas_call(
        functools.partial(_embed_body, offs),
        grid=(N // blk,),
        in_specs=[
            pl.BlockSpec((blk, 1), lambda i: (i, 0)),
            pl.BlockSpec((blk, 9), lambda i: (i, 0)),
            pl.BlockSpec((TAB, H), lambda i: (0, 0)),
            pl.BlockSpec((H, H), lambda i: (0, 0)),
            pl.BlockSpec((H, H), lambda i: (0, 0)),
            pl.BlockSpec((1, H), lambda i: (0, 0)),
        ],
        out_specs=[
            pl.BlockSpec((blk, H), lambda i: (i, 0)),
            pl.BlockSpec((blk, H), lambda i: (i, 0)),
        ],
        out_shape=[
            jax.ShapeDtypeStruct((N, H), F32),
            jax.ShapeDtypeStruct((N, H), F32),
        ],
    )(x2d, as2d, tab, wl, wr, b2)


def _gather_body(chunk, src_hbm, dst_hbm, cs_hbm, cd_hbm, csg_hbm, cdg_hbm,
                 idxs, idxd, ba0, ba1, bb0, bb1, sa0, sa1, sb0, sb1):
    cid = lax.axis_index("c")
    sid = lax.axis_index("s")
    wid = sid * 2 + cid
    lo = wid < 16
    r0 = chunk * 640 + jnp.where(lo, wid * 24, 384 + (wid - 16) * 16)
    nr = jnp.where(lo, 24, 16)
    out0 = jnp.where(lo, wid * 24, 384 + (wid - 16) * 16)

    @pl.when(lo)
    def _():
        pltpu.sync_copy(src_hbm.at[pl.ds(r0, 24), :], idxs)
        pltpu.sync_copy(dst_hbm.at[pl.ds(r0, 24), :], idxd)

    @pl.when(jnp.logical_not(lo))
    def _():
        pltpu.sync_copy(src_hbm.at[pl.ds(r0, 16), :],
                        idxs.at[pl.ds(0, 16), :])
        pltpu.sync_copy(dst_hbm.at[pl.ds(r0, 16), :],
                        idxd.at[pl.ds(0, 16), :])

    def start(r, ba, bb, sa, sb):
        pltpu.make_async_copy(cs_hbm.at[idxs.at[r]], ba, sa).start()
        pltpu.make_async_copy(cd_hbm.at[idxd.at[r]], bb, sb).start()

    def finish(r, ba, bb, sa, sb):
        pltpu.make_async_copy(cs_hbm.at[idxs.at[r]], ba, sa).wait()
        pltpu.make_async_copy(cd_hbm.at[idxd.at[r]], bb, sb).wait()
        e0 = (out0 + r) * 128
        pltpu.sync_copy(ba, csg_hbm.at[pl.ds(e0, 128), :])
        pltpu.sync_copy(bb, cdg_hbm.at[pl.ds(e0, 128), :])

    start(0, ba0, bb0, sa0, sb0)

    def pair(k, carry):
        start(2 * k + 1, ba1, bb1, sa1, sb1)
        finish(2 * k, ba0, bb0, sa0, sb0)
        start(2 * k + 2, ba0, bb0, sa0, sb0)
        finish(2 * k + 1, ba1, bb1, sa1, sb1)
        return carry

    lax.fori_loop(0, nr // 2 - 1, pair, 0)
    start(nr - 1, ba1, bb1, sa1, sb1)
    finish(nr - 2, ba0, bb0, sa0, sb0)
    finish(nr - 1, ba1, bb1, sa1, sb1)


def _sc_gather(chunk, cs, cd, src2d, dst2d):
    import functools
    f = pl.kernel(
        functools.partial(_gather_body, chunk),
        out_type=[
            jax.ShapeDtypeStruct((EC, H), F32),
            jax.ShapeDtypeStruct((EC, H), F32),
        ],
        mesh=plsc.VectorSubcoreMesh(core_axis_name="c", subcore_axis_name="s",
                                    num_cores=2, num_subcores=16),
        scratch_types=[
            pltpu.VMEM((24, 128), jnp.int32),
            pltpu.VMEM((24, 128), jnp.int32),
            pltpu.VMEM((128, H), F32),
            pltpu.VMEM((128, H), F32),
            pltpu.VMEM((128, H), F32),
            pltpu.VMEM((128, H), F32),
            pltpu.SemaphoreType.DMA,
            pltpu.SemaphoreType.DMA,
            pltpu.SemaphoreType.DMA,
            pltpu.SemaphoreType.DMA,
        ],
    )
    return f(src2d, dst2d, cs, cd)


def _edge_body(chunk, rbf_ref, csg_ref, cdg_ref, vt_ref, w1_ref, b1_ref,
               w2_ref, b2_ref, w3_ref, b3_ref, msg_ref):
    def compute():
        c = csg_ref[...] + cdg_ref[...]
        rbf = rbf_ref[...]
        dn = (((1,), (1,)), ((), ()))
        u1 = (lax.dot_general(rbf, w1_ref[...], dn,
                              preferred_element_type=F32) + b1_ref[...]) * c
        u2 = (lax.dot_general(rbf, w2_ref[...], dn,
                              preferred_element_type=F32) + b2_ref[...]) * c
        u3 = (lax.dot_general(rbf, w3_ref[...], dn,
                              preferred_element_type=F32) + b3_ref[...]) * c
        eye3 = jnp.eye(3, dtype=F32)
        vv = lax.dot_general(vt_ref[...], eye3, (((0,), (0,)), ((), ())),
                             preferred_element_type=F32, precision=HIGH)
        v0, v1, v2 = vv[:, 0:1], vv[:, 1:2], vv[:, 2:3]
        msg_ref[0] = u1
        msg_ref[1] = u2 * v0
        msg_ref[2] = u2 * v1
        msg_ref[3] = u2 * v2
        msg_ref[4] = u3 * (v0 * v0 - v2 * v2)
        msg_ref[5] = u3 * (v1 * v1 - v2 * v2)
        msg_ref[6] = u3 * (v0 * v1)
        msg_ref[7] = u3 * (v0 * v2)
        msg_ref[8] = u3 * (v1 * v2)

    if chunk == 0:
        compute()
    else:
        @pl.when(pl.program_id(0) < NBLK - 128)
        def _():
            compute()

        @pl.when(pl.program_id(0) >= NBLK - 128)
        def _():
            z = jnp.zeros((msg_ref.shape[1], H), F32)
            for k in range(NG):
                msg_ref[k] = z


def _edge_msg(chunk, rbf, csg, cdg, vt, w1, b1, w2, b2, w3, b3):
    blk = 640
    off = chunk * 128
    import functools
    wspec = pl.BlockSpec((H, R), lambda i: (0, 0))
    bspec = pl.BlockSpec((1, H), lambda i: (0, 0))
    return pl.pallas_call(
        functools.partial(_edge_body, chunk),
        grid=(EC // blk,),
        in_specs=[
            pl.BlockSpec((blk, R),
                         lambda i: (jnp.minimum(i + off, NBLK - 1), 0)),
            pl.BlockSpec((blk, H), lambda i: (i, 0)),
            pl.BlockSpec((blk, H), lambda i: (i, 0)),
            pl.BlockSpec((3, blk), lambda i: (0, i + off)),
            wspec, bspec, wspec, bspec, wspec, bspec,
        ],
        out_specs=pl.BlockSpec((NG, blk, H), lambda i: (0, i, 0)),
        out_shape=jax.ShapeDtypeStruct((NG, EC, H), F32),
    )(rbf, csg, cdg, vt, w1, b1, w2, b2, w3, b3)


def _scatter_body(chunk, dst_hbm, msg_hbm, p_hbm,
                  acc, idx, idx4, zbuf, b0, b1, s0, s1):
    cid = lax.axis_index("c")
    sid = lax.axis_index("s")
    r0 = chunk * 640 + sid * 40

    z16 = jnp.zeros((16,), F32)

    def zrow(r, carry):
        for cc in range(8):
            zbuf[r, pl.ds(cc * 16, 16)] = z16
        return carry

    lax.fori_loop(0, 64, zrow, 0)

    pltpu.sync_copy(dst_hbm.at[pl.ds(r0, 40), :], idx)

    @pl.when(cid == 0)
    def _():
        pltpu.sync_copy(dst_hbm.at[pl.ds(chunk * 640 + sid * 24, 24), :],
                        idx4)

    @pl.when(cid == 1)
    def _():
        pltpu.sync_copy(
            dst_hbm.at[pl.ds(chunk * 640 + 384 + sid * 16, 16), :],
            idx4.at[pl.ds(0, 16), :])

    def zero_acc():
        for j in range(9):
            pltpu.sync_copy(zbuf, acc.at[pl.ds(sid * 624 + j * 64, 64), :])
        pltpu.sync_copy(zbuf.at[pl.ds(0, 48), :],
                        acc.at[pl.ds(sid * 624 + 576, 48), :])

        @pl.when(sid == 15)
        def _():
            pltpu.sync_copy(zbuf.at[pl.ds(0, 16), :],
                            acc.at[pl.ds(9984, 16), :])

    def dump(gout):
        pltpu.sync_copy(acc.at[pl.ds(sid * 624, 624), :],
                        p_hbm.at[gout, pl.ds(sid * 624, 624), :])

        @pl.when(sid == 15)
        def _():
            pltpu.sync_copy(acc.at[pl.ds(9984, 16), :],
                            p_hbm.at[gout, pl.ds(9984, 16), :])

    def proc(g, gout, base, nb, idxbuf):
        zero_acc()
        plsc.subcore_barrier()

        def start(r, buf, sem):
            e0 = (base + r) * 128
            pltpu.make_async_copy(msg_hbm.at[g, pl.ds(e0, 128), :],
                                  buf, sem).start()

        def finish(r, buf, sem):
            e0 = (base + r) * 128
            pltpu.make_async_copy(msg_hbm.at[g, pl.ds(e0, 128), :],
                                  buf, sem).wait()
            pltpu.sync_copy(buf, acc.at[idxbuf.at[r]], add=True)

        start(0, b0, s0)

        def pair(k, carry):
            start(2 * k + 1, b1, s1)
            finish(2 * k, b0, s0)
            start(2 * k + 2, b0, s0)
            finish(2 * k + 1, b1, s1)
            return carry

        lax.fori_loop(0, nb // 2 - 1, pair, 0)
        start(nb - 1, b1, s1)
        finish(nb - 2, b0, s0)
        finish(nb - 1, b1, s1)

        plsc.subcore_barrier()
        dump(gout)
        plsc.subcore_barrier()

    @pl.when(cid == 0)
    def _():
        for g in (0, 1, 2, 3):
            proc(g, g, sid * 40, 40, idx)
        proc(4, 4, sid * 24, 24, idx4)

    @pl.when(cid == 1)
    def _():
        for g in (5, 6, 7, 8):
            proc(g, g, sid * 40, 40, idx)
        proc(4, 9, 384 + sid * 16, 16, idx4)


def _sc_scatter(chunk, msg, dst2d):
    import functools
    f = pl.kernel(
        functools.partial(_scatter_body, chunk),
        out_type=jax.ShapeDtypeStruct((NG + 1, N, H), F32),
        mesh=plsc.VectorSubcoreMesh(core_axis_name="c", subcore_axis_name="s",
                                    num_cores=2, num_subcores=16),
        scratch_types=[
            pltpu.VMEM_SHARED((N, H), F32),
            pltpu.VMEM((40, 128), jnp.int32),
            pltpu.VMEM((24, 128), jnp.int32),
            pltpu.VMEM((64, H), F32),
            pltpu.VMEM((128, H), F32),
            pltpu.VMEM((128, H), F32),
            pltpu.SemaphoreType.DMA,
            pltpu.SemaphoreType.DMA,
        ],
    )
    return f(dst2d, msg)


def _post_body(p0_ref, p1_ref, lng_ref, lnb_ref, ws1_ref, ws1b_ref,
               ws2r_ref, b20_ref, b21_ref, b22_ref,
               wt0_ref, wt1_ref, wt2_ref, out_ref):
    p0 = p0_ref[0] + p1_ref[0]
    pa0 = p0_ref[1] + p1_ref[1]
    pa1 = p0_ref[2] + p1_ref[2]
    pa2 = p0_ref[3] + p1_ref[3]
    pt0 = p0_ref[4] + p1_ref[4] + p0_ref[9] + p1_ref[9]
    pt1 = p0_ref[5] + p1_ref[5]
    p01 = p0_ref[6] + p1_ref[6]
    p02 = p0_ref[7] + p1_ref[7]
    p12 = p0_ref[8] + p1_ref[8]

    s00 = (2.0 * pt0 - pt1) / 3.0
    s11 = (2.0 * pt1 - pt0) / 3.0
    s22 = -(pt0 + pt1) / 3.0
    fro = (3.0 * p0 * p0
           + 2.0 * (pa0 * pa0 + pa1 * pa1 + pa2 * pa2)
           + s00 * s00 + s11 * s11 + s22 * s22
           + 2.0 * (p01 * p01 + p02 * p02 + p12 * p12))

    mu = jnp.mean(fro, axis=-1, keepdims=True)
    var = jnp.mean((fro - mu) ** 2, axis=-1, keepdims=True)
    y = lng_ref[...] * (fro - mu) * lax.rsqrt(var + 1e-5) + lnb_ref[...]

    dn = (((1,), (1,)), ((), ()))

    def silu(t):
        return t * lax.logistic(t)

    h1 = silu(lax.dot_general(y, ws1_ref[...], dn, preferred_element_type=F32)
              + ws1b_ref[...])
    w2all = ws2r_ref[...]
    w20 = w2all[:, 0, :]
    w21 = w2all[:, 1, :]
    w22 = w2all[:, 2, :]
    n0 = silu(lax.dot_general(h1, w20, dn, preferred_element_type=F32)
              + b20_ref[...])
    n1 = silu(lax.dot_general(h1, w21, dn, preferred_element_type=F32)
              + b21_ref[...])
    n2 = silu(lax.dot_general(h1, w22, dn, preferred_element_type=F32)
              + b22_ref[...])

    a = lax.dot_general(p0, wt0_ref[...], dn, preferred_element_type=F32) * n0
    wt1 = wt1_ref[...]
    b0 = lax.dot_general(pa0, wt1, dn, preferred_element_type=F32) * n1
    b1 = lax.dot_general(pa1, wt1, dn, preferred_element_type=F32) * n1
    b2 = lax.dot_general(pa2, wt1, dn, preferred_element_type=F32) * n1
    wt2 = wt2_ref[...]
    t0 = lax.dot_general(pt0, wt2, dn, preferred_element_type=F32) * n2
    t1 = lax.dot_general(pt1, wt2, dn, preferred_element_type=F32) * n2
    q01 = lax.dot_general(p01, wt2, dn, preferred_element_type=F32) * n2
    q02 = lax.dot_general(p02, wt2, dn, preferred_element_type=F32) * n2
    q12 = lax.dot_general(p12, wt2, dn, preferred_element_type=F32) * n2

    o00 = (2.0 * t0 - t1) / 3.0
    o11 = (2.0 * t1 - t0) / 3.0
    o22 = -(t0 + t1) / 3.0
    out_ref[0] = a + o00
    out_ref[1] = -b2 + q01
    out_ref[2] = b1 + q02
    out_ref[3] = b2 + q01
    out_ref[4] = a + o11
    out_ref[5] = -b0 + q12
    out_ref[6] = -b1 + q02
    out_ref[7] = b0 + q12
    out_ref[8] = a + o22


def _node_post(pp0, pp1, ln_g, ln_b, ws1, ws1b, ws2r, b20, b21, b22,
               wt0, wt1, wt2):
    blk = 1000
    hh = pl.BlockSpec((H, H), lambda i: (0, 0))
    bias = pl.BlockSpec((1, H), lambda i: (0, 0))
    pspec = pl.BlockSpec((NG + 1, blk, H), lambda i: (0, i, 0))

    return pl.pallas_call(
        _post_body,
        grid=(N // blk,),
        in_specs=[
            pspec, pspec,
            bias, bias,
            pl.BlockSpec((2 * H, H), lambda i: (0, 0)),
            pl.BlockSpec((1, 2 * H), lambda i: (0, 0)),
            pl.BlockSpec((H, 3, 2 * H), lambda i: (0, 0, 0)),
            bias, bias, bias,
            hh, hh, hh,
        ],
        out_specs=pl.BlockSpec((NG, blk, H), lambda i: (0, i, 0)),
        out_shape=jax.ShapeDtypeStruct((NG, N, H), F32),
    )(pp0, pp1, ln_g, ln_b, ws1, ws1b,
      ws2r, b20, b21, b22, wt0, wt1, wt2)


def kernel(x, atom_scalar, edge_index, dist, vec_norm, rbf, emb,
           ae0, ae1, ae2, ae3, ae4, ae5, ae6, ae7, ae8,
           W1, b1, W2, b2, W3, b3, emb2_W, emb2_b, ln_g, ln_b,
           Wt0, Wt1, Wt2, Ws1_W, Ws1_b, Ws2_W, Ws2_b):
    del dist

    aes = [ae0, ae1, ae2, ae3, ae4, ae5, ae6, ae7, ae8]
    dims = [a.shape[0] for a in aes]
    tab = jnp.concatenate(
        [emb] + aes + [jnp.zeros((TAB - 128 - sum(dims), H), F32)], axis=0)
    offs, o = [], 128
    for d in dims:
        offs.append(o)
        o += d

    wl = emb2_W[:, :H]
    wr = emb2_W[:, H:]
    fill = jnp.broadcast_to(
        jnp.arange(EP - E, dtype=jnp.int32)[None, :] % N, (2, EP - E))
    ei = jnp.concatenate([edge_index.astype(jnp.int32), fill], axis=1)
    src2d = ei[0].reshape(EROWSP, 128)
    dst2d = ei[1].reshape(EROWSP, 128)
    vt = jnp.pad(vec_norm.T, ((0, 0), (0, EP - E)))

    cs, cd = _embed(offs, x.astype(jnp.int32).reshape(N, 1),
                    atom_scalar.astype(jnp.int32), tab, wl, wr,
                    emb2_b.reshape(1, H))
    csg0, cdg0 = _sc_gather(0, cs, cd, src2d, dst2d)
    csg1, cdg1 = _sc_gather(1, cs, cd, src2d, dst2d)
    bias1 = (b1.reshape(1, H), b2.reshape(1, H), b3.reshape(1, H))
    msg0 = _edge_msg(0, rbf, csg0, cdg0, vt, W1, bias1[0], W2, bias1[1],
                     W3, bias1[2])
    msg1 = _edge_msg(1, rbf, csg1, cdg1, vt, W1, bias1[0], W2, bias1[1],
                     W3, bias1[2])
    pp0 = _sc_scatter(0, msg0, dst2d)
    pp1 = _sc_scatter(1, msg1, dst2d)
    out9 = _node_post(pp0, pp1, ln_g.reshape(1, H), ln_b.reshape(1, H),
                      Ws1_W, Ws1_b.reshape(1, 2 * H),
                      Ws2_W.reshape(H, 3, 2 * H),
                      Ws2_b[0::3].reshape(1, H),
                      Ws2_b[1::3].reshape(1, H),
                      Ws2_b[2::3].reshape(1, H),
                      Wt0, Wt1, Wt2)
    return jnp.transpose(out9, (1, 2, 0)).reshape(N, H, 3, 3)
